# baseline (speedup 1.0000x reference)
import functools

import jax
import jax.numpy as jnp
from jax.experimental import pallas as pl
from jax.experimental.pallas import tpu as pltpu
from jax.experimental.pallas import tpu_sc as plsc

E = 8
EC = 2
VS = E // EC
TS = 1024
NW = 32


def _mm_kernel(x_ref, ew_ref, gw_ref, eb_ref, aw_ref, ab_ref, ow_ref,
               ob_ref, out_ref, res_ref, wrows_s, b16_s, owt_s):
    s = pl.program_id(0)
    aw = aw_ref[...]

    @pl.when(s == 0)
    def _():
        wrows_s[0:E, :] = gw_ref[...]
        c = jax.lax.dot_general(
            aw, eb_ref[...], (((1,), (1,)), ((), ())),
            preferred_element_type=jnp.float32)
        b16_s[...] = jnp.concatenate(
            [jnp.zeros((1, E), jnp.float32), c + ab_ref[...]], axis=1)
        owt_s[...] = jnp.transpose(ow_ref[...]).astype(jnp.bfloat16)

    @pl.when(s < VS)
    def _():
        for j in range(EC):
            v = jnp.dot(aw, ew_ref[j], preferred_element_type=jnp.float32)
            wrows_s[pl.ds(E + s * EC + j, 1), :] = v

    @pl.when(s >= VS)
    def _():
        xt = x_ref[...]
        res = jax.lax.dot_general(
            xt, wrows_s[...], (((1,), (1,)), ((), ())),
            preferred_element_type=jnp.float32)
        res_ref[...] = jnp.transpose(res + b16_s[...])
        orig = jnp.dot(xt.astype(jnp.bfloat16), owt_s[...],
                       preferred_element_type=jnp.float32)
        out_ref[...] = orig + ob_ref[...]


def _add_kernel(noagg_ref, agg_ref, out_ref):
    out_ref[...] = noagg_ref[...] + agg_ref[...]


def kernel(x, gate_w, expert_w, expert_b, agg_w, agg_b, orig_w, orig_b):
    B, S, D = x.shape
    Ev, F, _ = expert_w.shape
    T = B * S
    CH = T // NW

    noagg, res16 = pl.pallas_call(
        _mm_kernel,
        grid=(VS + T // TS,),
        in_specs=[
            pl.BlockSpec((TS, D), lambda s: (jnp.maximum(s - VS, 0), 0)),
            pl.BlockSpec((EC, F, D), lambda s: (jnp.minimum(s, VS - 1), 0, 0)),
            pl.BlockSpec((Ev, D), lambda s: (0, 0)),
            pl.BlockSpec((Ev, F), lambda s: (0, 0)),
            pl.BlockSpec((1, F), lambda s: (0, 0)),
            pl.BlockSpec((1, 1), lambda s: (0, 0)),
            pl.BlockSpec((F, D), lambda s: (0, 0)),
            pl.BlockSpec((1, F), lambda s: (0, 0)),
        ],
        out_specs=[
            pl.BlockSpec((TS, F), lambda s: (jnp.maximum(s - VS, 0), 0)),
            pl.BlockSpec((2 * E, TS), lambda s: (0, jnp.maximum(s - VS, 0))),
        ],
        out_shape=[
            jax.ShapeDtypeStruct((T, F), jnp.float32),
            jax.ShapeDtypeStruct((2 * E, T), jnp.float32),
        ],
        scratch_shapes=[
            pltpu.VMEM((2 * E, D), jnp.float32),
            pltpu.VMEM((1, 2 * E), jnp.float32),
            pltpu.VMEM((D, F), jnp.bfloat16),
        ],
    )(x.reshape(T, D), expert_w, gate_w, expert_b, agg_w,
      agg_b.reshape(1, 1), orig_w, orig_b.reshape(1, F))

    @functools.partial(
        pl.kernel,
        mesh=plsc.VectorSubcoreMesh(core_axis_name="c", subcore_axis_name="s"),
        out_type=jax.ShapeDtypeStruct((T,), jnp.float32),
        scratch_types=[
            pltpu.VMEM((2 * E, CH), jnp.float32),
            pltpu.VMEM((CH,), jnp.float32),
        ],
    )
    def _sc_gate(res_hbm, agg_hbm, res_v, agg_v):
        wid = jax.lax.axis_index("s") * 2 + jax.lax.axis_index("c")
        base = wid * CH
        pltpu.sync_copy(res_hbm.at[:, pl.ds(base, CH)], res_v)
        neg = jnp.float32(-1e30)
        for b in range(CH // 16):
            cols = [
                res_v[e, pl.ds(16 * b, 16)]
                for e in range(2 * E)
            ]
            lg, sc = cols[:E], cols[E:]
            m1 = functools.reduce(jnp.maximum, lg)
            m2 = functools.reduce(
                jnp.maximum, [jnp.where(l < m1, l, neg) for l in lg])
            num = functools.reduce(
                jnp.add,
                [jnp.where(l >= m2, jnp.exp(l - m1) * s_, 0.0)
                 for l, s_ in zip(lg, sc)])
            den = 1.0 + jnp.exp(m2 - m1)
            agg_v[pl.ds(16 * b, 16)] = num / den
        pltpu.sync_copy(agg_v, agg_hbm.at[pl.ds(base, CH)])

    agg = _sc_gate(res16)

    out = pl.pallas_call(
        _add_kernel,
        grid=(T // TS,),
        in_specs=[
            pl.BlockSpec((TS, F), lambda s: (s, 0)),
            pl.BlockSpec((TS, 1), lambda s: (s, 0)),
        ],
        out_specs=pl.BlockSpec((TS, F), lambda s: (s, 0)),
        out_shape=jax.ShapeDtypeStruct((T, F), jnp.float32),
    )(noagg, agg.reshape(T, 1))

    return out.reshape(B, S, F)


# speedup vs baseline: 1.8771x; 1.8771x over previous
import jax
import jax.numpy as jnp
from jax.experimental import pallas as pl
from jax.experimental.pallas import tpu as pltpu

E = 8
EC = 2
VS = E // EC
TS = 1024


def _fused_kernel(x_ref, ew_ref, gw_ref, eb_ref, aw_ref, ab_ref, ow_ref,
                  ob_ref, out_ref, wrows_s, b16_s, owt_s):
    s = pl.program_id(0)
    aw = aw_ref[...]

    @pl.when(s == 0)
    def _():
        wrows_s[0:E, :] = gw_ref[...]
        c = jax.lax.dot_general(
            aw, eb_ref[...], (((1,), (1,)), ((), ())),
            preferred_element_type=jnp.float32)
        b16_s[...] = jnp.concatenate(
            [jnp.zeros((1, E), jnp.float32), c + ab_ref[...]], axis=1)
        owt_s[...] = jnp.transpose(ow_ref[...]).astype(jnp.bfloat16)

    @pl.when(s < VS)
    def _():
        for j in range(EC):
            v = jnp.dot(aw, ew_ref[j], preferred_element_type=jnp.float32)
            wrows_s[pl.ds(E + s * EC + j, 1), :] = v

    @pl.when(s >= VS)
    def _():
        xt = x_ref[...]
        res = jax.lax.dot_general(
            xt, wrows_s[...], (((1,), (1,)), ((), ())),
            preferred_element_type=jnp.float32)
        res = res + b16_s[...]
        lg = res[:, 0:E]
        sc = res[:, E:2 * E]
        m1 = jnp.max(lg, axis=1, keepdims=True)
        neg = jnp.float32(-jnp.inf)
        m2 = jnp.max(jnp.where(lg < m1, lg, neg), axis=1, keepdims=True)
        p = jnp.exp(lg - m1)
        num = jnp.sum(jnp.where(lg >= m2, p * sc, 0.0), axis=1, keepdims=True)
        den = 1.0 + jnp.exp(m2 - m1)
        agg = num / den

        orig = jnp.dot(xt.astype(jnp.bfloat16), owt_s[...],
                       preferred_element_type=jnp.float32)
        out_ref[...] = orig + ob_ref[...] + agg


def kernel(x, gate_w, expert_w, expert_b, agg_w, agg_b, orig_w, orig_b):
    B, S, D = x.shape
    Ev, F, _ = expert_w.shape
    T = B * S

    out = pl.pallas_call(
        _fused_kernel,
        grid=(VS + T // TS,),
        in_specs=[
            pl.BlockSpec((TS, D), lambda s: (jnp.maximum(s - VS, 0), 0)),
            pl.BlockSpec((EC, F, D), lambda s: (jnp.minimum(s, VS - 1), 0, 0)),
            pl.BlockSpec((Ev, D), lambda s: (0, 0)),
            pl.BlockSpec((Ev, F), lambda s: (0, 0)),
            pl.BlockSpec((1, F), lambda s: (0, 0)),
            pl.BlockSpec((1, 1), lambda s: (0, 0)),
            pl.BlockSpec((F, D), lambda s: (0, 0)),
            pl.BlockSpec((1, F), lambda s: (0, 0)),
        ],
        out_specs=pl.BlockSpec((TS, F), lambda s: (jnp.maximum(s - VS, 0), 0)),
        out_shape=jax.ShapeDtypeStruct((T, F), jnp.float32),
        scratch_shapes=[
            pltpu.VMEM((2 * E, D), jnp.float32),
            pltpu.VMEM((1, 2 * E), jnp.float32),
            pltpu.VMEM((D, F), jnp.bfloat16),
        ],
    )(x.reshape(T, D), expert_w, gate_w, expert_b, agg_w,
      agg_b.reshape(1, 1), orig_w, orig_b.reshape(1, F))

    return out.reshape(B, S, F)
